# Initial kernel scaffold; baseline (speedup 1.0000x reference)
import jax
import jax.numpy as jnp
from jax.experimental import pallas as pl


def kernel(feat, coord, grid_coord, scores, batch, W, gamma, beta):
    raise NotImplementedError("write your pallas kernel here")


# speedup vs baseline: 4.9156x; 4.9156x over previous
import functools

import jax
import jax.numpy as jnp
from jax import lax
from jax.experimental import pallas as pl

N = 262144
C = 128
STRIDE = 2
CODE_DEPTH = 10
NUM_BUCKETS = 8
NUM_BATCH = 8
G = 32
M = NUM_BATCH * G * G * G * NUM_BUCKETS
KEYS_PER_BATCH = G * G * G * NUM_BUCKETS

_ROWS = 2048


def _key_body(gx, gy, gz, b, s, key_out):
    nx = gx[...] >> 1
    ny = gy[...] >> 1
    nz = gz[...] >> 1
    bucket = jnp.clip((s[...] * NUM_BUCKETS).astype(jnp.int32), 0, NUM_BUCKETS - 1)
    k = ((b[...] * G + nx) * G + ny) * G + nz
    key_out[...] = k * NUM_BUCKETS + bucket


def _compute_keys(gx, gy, gz, b, s):
    return pl.pallas_call(
        _key_body,
        out_shape=jax.ShapeDtypeStruct((_ROWS, C), jnp.int32),
    )(gx, gy, gz, b, s)


def _erf(x):
    a1, a2, a3, a4, a5 = 0.254829592, -0.284496736, 1.421413741, -1.453152027, 1.061405429
    p = 0.3275911
    ax = jnp.abs(x)
    t = 1.0 / (1.0 + p * ax)
    poly = ((((a5 * t + a4) * t + a3) * t + a2) * t + a1) * t
    y = 1.0 - poly * jnp.exp(-ax * ax)
    return jnp.sign(x) * y


def _dense_body(sums, ssums, counts, w, gamma, beta, feat_out, score_out):
    cnt = counts[...]
    valid = cnt > 0.0
    denom = jnp.maximum(cnt, 1.0)
    pooled = sums[...] / denom
    h = lax.dot_general(pooled, w[...], (((1,), (1,)), ((), ())),
                        preferred_element_type=jnp.float32)
    mu = jnp.mean(h, axis=1, keepdims=True)
    hc = h - mu
    var = jnp.mean(hc * hc, axis=1, keepdims=True)
    hn = hc * lax.rsqrt(var + 1e-5) * gamma[...] + beta[...]
    g = 0.5 * hn * (1.0 + _erf(hn * 0.7071067811865476))
    feat_out[...] = jnp.where(valid, g, 0.0)
    score_out[...] = jnp.where(valid, ssums[...] / denom, 0.0)


def _dense_stage(sums, ssums, counts, w, gamma, beta):
    blk = 2048
    grid = N // blk
    return pl.pallas_call(
        _dense_body,
        grid=(grid,),
        in_specs=[
            pl.BlockSpec((blk, C), lambda i: (i, i * 0)),
            pl.BlockSpec((blk, 1), lambda i: (i, i * 0)),
            pl.BlockSpec((blk, 1), lambda i: (i, i * 0)),
            pl.BlockSpec((C, C), lambda i: (i * 0, i * 0)),
            pl.BlockSpec((1, C), lambda i: (i * 0, i * 0)),
            pl.BlockSpec((1, C), lambda i: (i * 0, i * 0)),
        ],
        out_specs=[
            pl.BlockSpec((blk, C), lambda i: (i, i * 0)),
            pl.BlockSpec((blk, 1), lambda i: (i, i * 0)),
        ],
        out_shape=[
            jax.ShapeDtypeStruct((N, C), jnp.float32),
            jax.ShapeDtypeStruct((N, 1), jnp.float32),
        ],
    )(sums, ssums, counts, w, gamma, beta)


def kernel(feat, coord, grid_coord, scores, batch, W, gamma, beta):
    n = N
    gc32 = grid_coord.astype(jnp.int32)
    b32 = batch.astype(jnp.int32)
    gx = gc32[:, 0].reshape(_ROWS, C)
    gy = gc32[:, 1].reshape(_ROWS, C)
    gz = gc32[:, 2].reshape(_ROWS, C)
    brs = b32.reshape(_ROWS, C)
    srs = scores[:, 0].reshape(_ROWS, C)

    key = _compute_keys(gx, gy, gz, brs, srs).reshape(n)

    cnt = jnp.zeros((M,), jnp.int32).at[key].add(1)
    occ = (cnt > 0).astype(jnp.int32)
    inc = jnp.cumsum(occ)
    cluster = inc[key] - 1
    k_total = inc[M - 1]

    idx32 = jnp.arange(n, dtype=jnp.int32)
    minidx = jnp.full((M,), n, jnp.int32).at[key].min(idx32)
    is_rep = minidx[key] == idx32
    uix_core = jnp.zeros((n,), jnp.int32).at[cluster].add(
        jnp.where(is_rep, idx32, 0))
    uix = jnp.where(idx32 < k_total, uix_core, uix_core[0])

    counts = jnp.zeros((n,), jnp.float32).at[cluster].add(1.0)
    sums = jnp.zeros((n, C), jnp.float32).at[cluster].add(feat)
    ssums = jnp.zeros((n,), jnp.float32).at[cluster].add(scores[:, 0])

    new_feat, new_scores = _dense_stage(
        sums, ssums.reshape(n, 1), counts.reshape(n, 1),
        W.astype(jnp.float32), gamma.astype(jnp.float32).reshape(1, C),
        beta.astype(jnp.float32).reshape(1, C))

    new_feat = new_feat.astype(jnp.float64)

    new_grid = (grid_coord // STRIDE)[uix]
    new_coord = coord[uix]
    new_batch = batch[uix]

    boundaries = (jnp.arange(NUM_BATCH, dtype=jnp.int32) + 1) * KEYS_PER_BATCH - 1
    offset = inc[boundaries].astype(jnp.int64)

    seq = jnp.arange(n, dtype=jnp.int64)
    new_code = (new_batch.astype(jnp.int64) << (CODE_DEPTH * 3)) + seq
    new_order = seq[None, :]
    new_inverse = seq[None, :]
    return (new_feat, new_scores, new_grid, new_coord, new_batch, offset,
            cluster.astype(jnp.int64), new_code[None, :], new_order, new_inverse)


# speedup vs baseline: 13.0825x; 2.6614x over previous
import functools

import jax
import jax.numpy as jnp
from jax import lax
from jax.experimental import pallas as pl

N = 262144
C = 128
STRIDE = 2
CODE_DEPTH = 10
NUM_BUCKETS = 8
NUM_BATCH = 8
G = 32
M = NUM_BATCH * G * G * G * NUM_BUCKETS
KEYS_PER_BATCH = G * G * G * NUM_BUCKETS

_ROWS = 2048
CW = 136


def _key_body(gx, gy, gz, b, s, key_out):
    nx = gx[...] >> 1
    ny = gy[...] >> 1
    nz = gz[...] >> 1
    bucket = jnp.clip((s[...] * NUM_BUCKETS).astype(jnp.int32), 0, NUM_BUCKETS - 1)
    k = ((b[...] * G + nx) * G + ny) * G + nz
    key_out[...] = k * NUM_BUCKETS + bucket


def _compute_keys(gx, gy, gz, b, s):
    return pl.pallas_call(
        _key_body,
        out_shape=jax.ShapeDtypeStruct((_ROWS, C), jnp.int32),
    )(gx, gy, gz, b, s)


def _erf(x):
    a1, a2, a3, a4, a5 = 0.254829592, -0.284496736, 1.421413741, -1.453152027, 1.061405429
    p = 0.3275911
    ax = jnp.abs(x)
    t = 1.0 / (1.0 + p * ax)
    poly = ((((a5 * t + a4) * t + a3) * t + a2) * t + a1) * t
    y = 1.0 - poly * jnp.exp(-ax * ax)
    return jnp.sign(x) * y


def _dense_body(comb, w, gamma, beta, feat_out, score_out):
    sums = comb[:, :C]
    ssums = comb[:, C:C + 1]
    cnt = comb[:, C + 1:C + 2]
    valid = cnt > 0.0
    denom = jnp.maximum(cnt, 1.0)
    pooled = sums / denom
    h = lax.dot_general(pooled, w[...], (((1,), (1,)), ((), ())),
                        preferred_element_type=jnp.float32)
    mu = jnp.mean(h, axis=1, keepdims=True)
    hc = h - mu
    var = jnp.mean(hc * hc, axis=1, keepdims=True)
    hn = hc * lax.rsqrt(var + 1e-5) * gamma[...] + beta[...]
    g = 0.5 * hn * (1.0 + _erf(hn * 0.7071067811865476))
    feat_out[...] = jnp.where(valid, g, 0.0)
    score_out[...] = jnp.where(valid, ssums / denom, 0.0)


def _dense_stage(comb, w, gamma, beta):
    blk = 2048
    grid = N // blk
    return pl.pallas_call(
        _dense_body,
        grid=(grid,),
        in_specs=[
            pl.BlockSpec((blk, CW), lambda i: (i, i * 0)),
            pl.BlockSpec((C, C), lambda i: (i * 0, i * 0)),
            pl.BlockSpec((1, C), lambda i: (i * 0, i * 0)),
            pl.BlockSpec((1, C), lambda i: (i * 0, i * 0)),
        ],
        out_specs=[
            pl.BlockSpec((blk, C), lambda i: (i, i * 0)),
            pl.BlockSpec((blk, 1), lambda i: (i, i * 0)),
        ],
        out_shape=[
            jax.ShapeDtypeStruct((N, C), jnp.float32),
            jax.ShapeDtypeStruct((N, 1), jnp.float32),
        ],
    )(comb, w, gamma, beta)


def kernel(feat, coord, grid_coord, scores, batch, W, gamma, beta):
    n = N
    gc32 = grid_coord.astype(jnp.int32)
    b32 = batch.astype(jnp.int32)
    gx = gc32[:, 0].reshape(_ROWS, C)
    gy = gc32[:, 1].reshape(_ROWS, C)
    gz = gc32[:, 2].reshape(_ROWS, C)
    brs = b32.reshape(_ROWS, C)
    srs = scores[:, 0].reshape(_ROWS, C)

    key = _compute_keys(gx, gy, gz, brs, srs).reshape(n)

    idx32 = jnp.arange(n, dtype=jnp.int32)
    minidx = jnp.full((M,), n, jnp.int32).at[key].min(idx32)
    inc = jnp.cumsum((minidx < n).astype(jnp.int32))
    k_total = inc[M - 1]

    table = jnp.stack([inc, minidx], axis=1)
    tg = table[key]
    cluster = tg[:, 0] - 1
    is_rep = tg[:, 1] == idx32

    rows = jnp.concatenate([
        feat,
        scores,
        jnp.ones((n, 1), jnp.float32),
        jnp.where(is_rep, idx32, 0).astype(jnp.float32)[:, None],
        jnp.zeros((n, CW - C - 3), jnp.float32),
    ], axis=1)
    comb = jnp.zeros((n, CW), jnp.float32).at[cluster].add(rows)

    uix_core = comb[:, C + 2].astype(jnp.int32)
    uix = jnp.where(idx32 < k_total, uix_core, uix_core[0])

    new_feat, new_scores = _dense_stage(
        comb,
        W.astype(jnp.float32), gamma.astype(jnp.float32).reshape(1, C),
        beta.astype(jnp.float32).reshape(1, C))

    new_feat = new_feat.astype(jnp.float64)

    attrs = jnp.concatenate([
        (gc32 >> 1),
        b32[:, None],
        lax.bitcast_convert_type(coord, jnp.int32),
    ], axis=1)
    ga = attrs[uix]
    new_grid = ga[:, :3].astype(jnp.int64)
    new_coord = lax.bitcast_convert_type(ga[:, 4:7], jnp.float32)
    new_batch = ga[:, 3].astype(jnp.int64)

    boundaries = (jnp.arange(NUM_BATCH, dtype=jnp.int32) + 1) * KEYS_PER_BATCH - 1
    offset = inc[boundaries].astype(jnp.int64)

    seq = jnp.arange(n, dtype=jnp.int64)
    new_code = (new_batch.astype(jnp.int64) << (CODE_DEPTH * 3)) + seq
    new_order = seq[None, :]
    new_inverse = seq[None, :]
    return (new_feat, new_scores, new_grid, new_coord, new_batch, offset,
            cluster.astype(jnp.int64), new_code[None, :], new_order, new_inverse)


# speedup vs baseline: 13.3659x; 1.0217x over previous
import functools

import jax
import jax.numpy as jnp
from jax import lax
from jax.experimental import pallas as pl
from jax.experimental.pallas import tpu as pltpu

N = 262144
C = 128
STRIDE = 2
CODE_DEPTH = 10
NUM_BUCKETS = 8
NUM_BATCH = 8
G = 32
M = NUM_BATCH * G * G * G * NUM_BUCKETS
KEYS_PER_BATCH = G * G * G * NUM_BUCKETS

_ROWS = 2048
CW = 136


def _key_body(gx, gy, gz, b, s, key_out):
    nx = gx[...] >> 1
    ny = gy[...] >> 1
    nz = gz[...] >> 1
    bucket = jnp.clip((s[...] * NUM_BUCKETS).astype(jnp.int32), 0, NUM_BUCKETS - 1)
    k = ((b[...] * G + nx) * G + ny) * G + nz
    key_out[...] = k * NUM_BUCKETS + bucket


def _compute_keys(gx, gy, gz, b, s):
    return pl.pallas_call(
        _key_body,
        out_shape=jax.ShapeDtypeStruct((_ROWS, C), jnp.int32),
    )(gx, gy, gz, b, s)


_PB = 1024
_PA = M // _PB
_PRB = 256


def _tri(nrow, ncol, strict):
    r = lax.broadcasted_iota(jnp.int32, (nrow, ncol), 0)
    c = lax.broadcasted_iota(jnp.int32, (nrow, ncol), 1)
    return (r < c if strict else r <= c).astype(jnp.float32)


def _prefix_body(minidx_ref, inc_ref, carry):
    @pl.when(pl.program_id(0) == 0)
    def _():
        carry[0] = 0.0

    x = (minidx_ref[...] < N).astype(jnp.float32)
    y = jnp.dot(x, _tri(_PB, _PB, strict=False),
                preferred_element_type=jnp.float32)
    r = y[:, _PB - 1:_PB]
    ri = lax.broadcasted_iota(jnp.int32, (_PRB, _PRB), 0)
    ci = lax.broadcasted_iota(jnp.int32, (_PRB, _PRB), 1)
    tl = (ci < ri).astype(jnp.float32)
    p = jnp.dot(tl, r, preferred_element_type=jnp.float32)
    c = carry[0]
    inc_ref[...] = (y + p + c).astype(jnp.int32)
    carry[0] = c + jnp.sum(x)


def _prefix_stage(minidx):
    return pl.pallas_call(
        _prefix_body,
        grid=(_PA // _PRB,),
        in_specs=[pl.BlockSpec((_PRB, _PB), lambda i: (i, i * 0))],
        out_specs=pl.BlockSpec((_PRB, _PB), lambda i: (i, i * 0)),
        out_shape=jax.ShapeDtypeStruct((_PA, _PB), jnp.int32),
        scratch_shapes=[pltpu.SMEM((1,), jnp.float32)],
    )(minidx.reshape(_PA, _PB))


def _erf(x):
    a1, a2, a3, a4, a5 = 0.254829592, -0.284496736, 1.421413741, -1.453152027, 1.061405429
    p = 0.3275911
    ax = jnp.abs(x)
    t = 1.0 / (1.0 + p * ax)
    poly = ((((a5 * t + a4) * t + a3) * t + a2) * t + a1) * t
    y = 1.0 - poly * jnp.exp(-ax * ax)
    return jnp.sign(x) * y


def _dense_body(comb, w, gamma, beta, feat_out, score_out):
    sums = comb[:, :C]
    ssums = comb[:, C:C + 1]
    cnt = comb[:, C + 1:C + 2]
    valid = cnt > 0.0
    denom = jnp.maximum(cnt, 1.0)
    pooled = sums / denom
    h = lax.dot_general(pooled, w[...], (((1,), (1,)), ((), ())),
                        preferred_element_type=jnp.float32)
    mu = jnp.mean(h, axis=1, keepdims=True)
    hc = h - mu
    var = jnp.mean(hc * hc, axis=1, keepdims=True)
    hn = hc * lax.rsqrt(var + 1e-5) * gamma[...] + beta[...]
    g = 0.5 * hn * (1.0 + _erf(hn * 0.7071067811865476))
    feat_out[...] = jnp.where(valid, g, 0.0)
    score_out[...] = jnp.where(valid, ssums / denom, 0.0)


def _dense_stage(comb, w, gamma, beta):
    blk = 2048
    grid = N // blk
    return pl.pallas_call(
        _dense_body,
        grid=(grid,),
        in_specs=[
            pl.BlockSpec((blk, CW), lambda i: (i, i * 0)),
            pl.BlockSpec((C, C), lambda i: (i * 0, i * 0)),
            pl.BlockSpec((1, C), lambda i: (i * 0, i * 0)),
            pl.BlockSpec((1, C), lambda i: (i * 0, i * 0)),
        ],
        out_specs=[
            pl.BlockSpec((blk, C), lambda i: (i, i * 0)),
            pl.BlockSpec((blk, 1), lambda i: (i, i * 0)),
        ],
        out_shape=[
            jax.ShapeDtypeStruct((N, C), jnp.float32),
            jax.ShapeDtypeStruct((N, 1), jnp.float32),
        ],
    )(comb, w, gamma, beta)


def kernel(feat, coord, grid_coord, scores, batch, W, gamma, beta):
    n = N
    gc32 = grid_coord.astype(jnp.int32)
    b32 = batch.astype(jnp.int32)
    gx = gc32[:, 0].reshape(_ROWS, C)
    gy = gc32[:, 1].reshape(_ROWS, C)
    gz = gc32[:, 2].reshape(_ROWS, C)
    brs = b32.reshape(_ROWS, C)
    srs = scores[:, 0].reshape(_ROWS, C)

    key = _compute_keys(gx, gy, gz, brs, srs).reshape(n)

    idx32 = jnp.arange(n, dtype=jnp.int32)
    minidx = jnp.full((M,), n, jnp.int32).at[key].min(idx32)
    inc = _prefix_stage(minidx).reshape(M)
    k_total = inc[M - 1]

    table = jnp.stack([inc, minidx], axis=1)
    tg = table[key]
    cluster = tg[:, 0] - 1
    is_rep = tg[:, 1] == idx32

    rows = jnp.concatenate([
        feat,
        scores,
        jnp.ones((n, 1), jnp.float32),
        jnp.where(is_rep, idx32, 0).astype(jnp.float32)[:, None],
        jnp.zeros((n, CW - C - 3), jnp.float32),
    ], axis=1)
    comb = jnp.zeros((n, CW), jnp.float32).at[cluster].add(rows)

    uix_core = comb[:, C + 2].astype(jnp.int32)
    uix = jnp.where(idx32 < k_total, uix_core, uix_core[0])

    new_feat, new_scores = _dense_stage(
        comb,
        W.astype(jnp.float32), gamma.astype(jnp.float32).reshape(1, C),
        beta.astype(jnp.float32).reshape(1, C))

    new_feat = new_feat.astype(jnp.float64)

    attrs = jnp.concatenate([
        (gc32 >> 1),
        b32[:, None],
        lax.bitcast_convert_type(coord, jnp.int32),
    ], axis=1)
    ga = attrs[uix]
    new_grid = ga[:, :3].astype(jnp.int64)
    new_coord = lax.bitcast_convert_type(ga[:, 4:7], jnp.float32)
    new_batch = ga[:, 3].astype(jnp.int64)

    boundaries = (jnp.arange(NUM_BATCH, dtype=jnp.int32) + 1) * KEYS_PER_BATCH - 1
    offset = inc[boundaries].astype(jnp.int64)

    seq = jnp.arange(n, dtype=jnp.int64)
    new_code = (new_batch.astype(jnp.int64) << (CODE_DEPTH * 3)) + seq
    new_order = seq[None, :]
    new_inverse = seq[None, :]
    return (new_feat, new_scores, new_grid, new_coord, new_batch, offset,
            cluster.astype(jnp.int64), new_code[None, :], new_order, new_inverse)


# speedup vs baseline: 15.1735x; 1.1352x over previous
import functools

import jax
import jax.numpy as jnp
from jax import lax
from jax.experimental import pallas as pl
from jax.experimental.pallas import tpu as pltpu

N = 262144
C = 128
STRIDE = 2
CODE_DEPTH = 10
NUM_BUCKETS = 8
NUM_BATCH = 8
G = 32
M = NUM_BATCH * G * G * G * NUM_BUCKETS
KEYS_PER_BATCH = G * G * G * NUM_BUCKETS

_ROWS = 2048
EW = 8


def _key_body(gx, gy, gz, b, s, key_out):
    nx = gx[...] >> 1
    ny = gy[...] >> 1
    nz = gz[...] >> 1
    bucket = jnp.clip((s[...] * NUM_BUCKETS).astype(jnp.int32), 0, NUM_BUCKETS - 1)
    k = ((b[...] * G + nx) * G + ny) * G + nz
    key_out[...] = k * NUM_BUCKETS + bucket


def _compute_keys(gx, gy, gz, b, s):
    return pl.pallas_call(
        _key_body,
        out_shape=jax.ShapeDtypeStruct((_ROWS, C), jnp.int32),
    )(gx, gy, gz, b, s)


_PB = 1024
_PA = M // _PB
_PRB = 256


def _tri(nrow, ncol, strict):
    r = lax.broadcasted_iota(jnp.int32, (nrow, ncol), 0)
    c = lax.broadcasted_iota(jnp.int32, (nrow, ncol), 1)
    return (r < c if strict else r <= c).astype(jnp.float32)


def _prefix_body(minidx_ref, inc_ref, carry):
    @pl.when(pl.program_id(0) == 0)
    def _():
        carry[0] = 0.0

    x = (minidx_ref[...] < N).astype(jnp.float32)
    y = jnp.dot(x, _tri(_PB, _PB, strict=False),
                preferred_element_type=jnp.float32)
    r = y[:, _PB - 1:_PB]
    ri = lax.broadcasted_iota(jnp.int32, (_PRB, _PRB), 0)
    ci = lax.broadcasted_iota(jnp.int32, (_PRB, _PRB), 1)
    tl = (ci < ri).astype(jnp.float32)
    p = jnp.dot(tl, r, preferred_element_type=jnp.float32)
    c = carry[0]
    inc_ref[...] = (y + p + c).astype(jnp.int32)
    carry[0] = c + jnp.sum(x)


def _prefix_stage(minidx):
    return pl.pallas_call(
        _prefix_body,
        grid=(_PA // _PRB,),
        in_specs=[pl.BlockSpec((_PRB, _PB), lambda i: (i, i * 0))],
        out_specs=pl.BlockSpec((_PRB, _PB), lambda i: (i, i * 0)),
        out_shape=jax.ShapeDtypeStruct((_PA, _PB), jnp.int32),
        scratch_shapes=[pltpu.SMEM((1,), jnp.float32)],
    )(minidx.reshape(_PA, _PB))


def _erf(x):
    a1, a2, a3, a4, a5 = 0.254829592, -0.284496736, 1.421413741, -1.453152027, 1.061405429
    p = 0.3275911
    ax = jnp.abs(x)
    t = 1.0 / (1.0 + p * ax)
    poly = ((((a5 * t + a4) * t + a3) * t + a2) * t + a1) * t
    y = 1.0 - poly * jnp.exp(-ax * ax)
    return jnp.sign(x) * y


def _dense_body(comb, extras, w, gamma, beta, feat_out, score_out):
    sums = comb[...]
    ssums = extras[:, 0:1]
    cnt = extras[:, 1:2]
    valid = cnt > 0.0
    denom = jnp.maximum(cnt, 1.0)
    pooled = sums / denom
    h = lax.dot_general(pooled, w[...], (((1,), (1,)), ((), ())),
                        preferred_element_type=jnp.float32)
    mu = jnp.mean(h, axis=1, keepdims=True)
    hc = h - mu
    var = jnp.mean(hc * hc, axis=1, keepdims=True)
    hn = hc * lax.rsqrt(var + 1e-5) * gamma[...] + beta[...]
    g = 0.5 * hn * (1.0 + _erf(hn * 0.7071067811865476))
    feat_out[...] = jnp.where(valid, g, 0.0)
    score_out[...] = jnp.where(valid, ssums / denom, 0.0)


def _dense_stage(comb, extras, w, gamma, beta):
    blk = 2048
    grid = N // blk
    return pl.pallas_call(
        _dense_body,
        grid=(grid,),
        in_specs=[
            pl.BlockSpec((blk, C), lambda i: (i, i * 0)),
            pl.BlockSpec((blk, EW), lambda i: (i, i * 0)),
            pl.BlockSpec((C, C), lambda i: (i * 0, i * 0)),
            pl.BlockSpec((1, C), lambda i: (i * 0, i * 0)),
            pl.BlockSpec((1, C), lambda i: (i * 0, i * 0)),
        ],
        out_specs=[
            pl.BlockSpec((blk, C), lambda i: (i, i * 0)),
            pl.BlockSpec((blk, 1), lambda i: (i, i * 0)),
        ],
        out_shape=[
            jax.ShapeDtypeStruct((N, C), jnp.float32),
            jax.ShapeDtypeStruct((N, 1), jnp.float32),
        ],
    )(comb, extras, w, gamma, beta)


def kernel(feat, coord, grid_coord, scores, batch, W, gamma, beta):
    n = N
    gc32 = grid_coord.astype(jnp.int32)
    b32 = batch.astype(jnp.int32)
    gx = gc32[:, 0].reshape(_ROWS, C)
    gy = gc32[:, 1].reshape(_ROWS, C)
    gz = gc32[:, 2].reshape(_ROWS, C)
    brs = b32.reshape(_ROWS, C)
    srs = scores[:, 0].reshape(_ROWS, C)

    key = _compute_keys(gx, gy, gz, brs, srs).reshape(n)

    idx32 = jnp.arange(n, dtype=jnp.int32)
    minidx = jnp.full((M,), n, jnp.int32).at[key].min(idx32)
    inc = _prefix_stage(minidx).reshape(M)
    k_total = inc[M - 1]

    table = jnp.stack([inc, minidx], axis=1)
    tg = table[key]
    cluster = tg[:, 0] - 1
    is_rep = tg[:, 1] == idx32

    comb = jnp.zeros((n, C), jnp.float32).at[cluster].add(feat)
    ex_rows = jnp.concatenate([
        scores,
        jnp.ones((n, 1), jnp.float32),
        jnp.where(is_rep, idx32, 0).astype(jnp.float32)[:, None],
        jnp.zeros((n, EW - 3), jnp.float32),
    ], axis=1)
    extras = jnp.zeros((n, EW), jnp.float32).at[cluster].add(ex_rows)

    uix_core = extras[:, 2].astype(jnp.int32)
    uix = jnp.where(idx32 < k_total, uix_core, uix_core[0])

    new_feat, new_scores = _dense_stage(
        comb, extras,
        W.astype(jnp.float32), gamma.astype(jnp.float32).reshape(1, C),
        beta.astype(jnp.float32).reshape(1, C))

    new_feat = new_feat.astype(jnp.float64)

    attrs = jnp.concatenate([
        (gc32 >> 1),
        b32[:, None],
        lax.bitcast_convert_type(coord, jnp.int32),
    ], axis=1)
    ga = attrs[uix]
    new_grid = ga[:, :3].astype(jnp.int64)
    new_coord = lax.bitcast_convert_type(ga[:, 4:7], jnp.float32)
    new_batch = ga[:, 3].astype(jnp.int64)

    boundaries = (jnp.arange(NUM_BATCH, dtype=jnp.int32) + 1) * KEYS_PER_BATCH - 1
    offset = inc[boundaries].astype(jnp.int64)

    seq = jnp.arange(n, dtype=jnp.int64)
    new_code = (new_batch.astype(jnp.int64) << (CODE_DEPTH * 3)) + seq
    new_order = seq[None, :]
    new_inverse = seq[None, :]
    return (new_feat, new_scores, new_grid, new_coord, new_batch, offset,
            cluster.astype(jnp.int64), new_code[None, :], new_order, new_inverse)
